# baseline (speedup 1.0000x reference)
import jax
import jax.numpy as jnp
from jax import lax
from jax.experimental import pallas as pl
from jax.experimental.pallas import tpu as pltpu
from jax.experimental.pallas import tpu_sc as plsc

_GRID = 32
_NPATCH = _GRID * _GRID
_L = 16
_NSUB = 16
_NCORE = 2
_OUT_PER_SUB = _NPATCH // (_NSUB * _NCORE)
_REGION_PTS = (128, 64, 128)


def _sc_body(lm_hbm, ew_hbm, nw_hbm, mw_hbm, out_hbm,
             lm_v, ew_v, nw_v, mw_v,
             m_e, m_n, m_m, out_v, sem_lm, sem_w):
    wid = lax.axis_index("c") * _NSUB + lax.axis_index("s")
    off = wid * _OUT_PER_SUB

    cp_lm = pltpu.async_copy(lm_hbm, lm_v, sem_lm)
    cp_w = [
        pltpu.async_copy(ew_hbm.at[pl.ds(off, _OUT_PER_SUB)], ew_v, sem_w),
        pltpu.async_copy(nw_hbm.at[pl.ds(off, _OUT_PER_SUB)], nw_v, sem_w),
        pltpu.async_copy(mw_hbm.at[pl.ds(off, _OUT_PER_SUB)], mw_v, sem_w),
    ]

    zeros_f = jnp.zeros((_L,), jnp.float32)

    def zero_body(j, _):
        sl = pl.ds(off + j * _L, _L)
        m_e[sl] = zeros_f
        m_n[sl] = zeros_f
        m_m[sl] = zeros_f
        return 0

    lax.fori_loop(0, _OUT_PER_SUB // _L, zero_body, 0, unroll=False)

    lane2 = lax.iota(jnp.int32, _L) * 2
    one_i = jnp.ones((_L,), jnp.int32)
    one_f = jnp.ones((_L,), jnp.float32)
    cap = jnp.full((_L,), _GRID - 1, jnp.int32)

    cp_lm.wait()

    def make_scatter_body(mask, base):
        def scatter_body(i, _):
            xi = base + i * (2 * _L) + lane2
            xs = plsc.load_gather(lm_v, [xi])
            ys = plsc.load_gather(lm_v, [xi + one_i])
            r = jnp.minimum(ys >> 4, cap)
            c = jnp.minimum(xs >> 4, cap)
            bins = (r << 5) | c
            plsc.store_scatter(mask, [bins], one_f)
            return 0
        return scatter_body

    base = 0
    for n_pts, mask in zip(_REGION_PTS, (m_e, m_n, m_m)):
        lax.fori_loop(0, n_pts // _L, make_scatter_body(mask, base), 0,
                      unroll=False)
        base += 2 * n_pts

    for cp in cp_w:
        cp.wait()

    def combine_body(j, _):
        sl = pl.ds(off + j * _L, _L)
        wl = pl.ds(j * _L, _L)
        acc = (ew_v[wl] - one_f) * m_e[sl] + one_f
        acc = acc + (nw_v[wl] - one_f) * m_n[sl]
        out_v[wl] = acc + (mw_v[wl] - one_f) * m_m[sl]
        return 0

    lax.fori_loop(0, _OUT_PER_SUB // _L, combine_body, 0, unroll=False)

    pltpu.sync_copy(out_v, out_hbm.at[pl.ds(off, _OUT_PER_SUB)])


def kernel(eye_landmarks, nose_landmarks, mouth_landmarks,
           eye_weight, nose_weight, mouth_weight):
    lm = jnp.concatenate([eye_landmarks.astype(jnp.int32).reshape(-1),
                          nose_landmarks.astype(jnp.int32).reshape(-1),
                          mouth_landmarks.astype(jnp.int32).reshape(-1)])

    mesh = plsc.VectorSubcoreMesh(core_axis_name="c", subcore_axis_name="s",
                                  num_cores=_NCORE)
    run = pl.kernel(
        _sc_body,
        out_type=jax.ShapeDtypeStruct((_NPATCH,), jnp.float32),
        mesh=mesh,
        compiler_params=pltpu.CompilerParams(needs_layout_passes=False),
        scratch_types=[
            pltpu.VMEM((2 * sum(_REGION_PTS),), jnp.int32),
            pltpu.VMEM((_OUT_PER_SUB,), jnp.float32),
            pltpu.VMEM((_OUT_PER_SUB,), jnp.float32),
            pltpu.VMEM((_OUT_PER_SUB,), jnp.float32),
            pltpu.VMEM((_NPATCH,), jnp.float32),
            pltpu.VMEM((_NPATCH,), jnp.float32),
            pltpu.VMEM((_NPATCH,), jnp.float32),
            pltpu.VMEM((_OUT_PER_SUB,), jnp.float32),
            pltpu.SemaphoreType.DMA,
            pltpu.SemaphoreType.DMA,
        ],
    )
    return run(lm, eye_weight, nose_weight, mouth_weight)


# speedup vs baseline: 1.1027x; 1.1027x over previous
import jax
import jax.numpy as jnp
from jax import lax
from jax.experimental import pallas as pl
from jax.experimental.pallas import tpu as pltpu
from jax.experimental.pallas import tpu_sc as plsc

_GRID = 32
_NPATCH = _GRID * _GRID
_L = 16
_NSUB = 16
_OUT_PER_SUB = _NPATCH // _NSUB
_REGION_PTS = (128, 64, 128)


def _sc_body(lm_hbm, ew_hbm, nw_hbm, mw_hbm, out_hbm,
             lm_v, ew_v, nw_v, mw_v,
             m_e, m_n, m_m, out_v, sem_lm, sem_w):
    s = lax.axis_index("s")
    off = s * _OUT_PER_SUB

    cp_lm = pltpu.async_copy(lm_hbm, lm_v, sem_lm)
    cp_w = [
        pltpu.async_copy(ew_hbm.at[pl.ds(off, _OUT_PER_SUB)], ew_v, sem_w),
        pltpu.async_copy(nw_hbm.at[pl.ds(off, _OUT_PER_SUB)], nw_v, sem_w),
        pltpu.async_copy(mw_hbm.at[pl.ds(off, _OUT_PER_SUB)], mw_v, sem_w),
    ]

    zeros_f = jnp.zeros((_L,), jnp.float32)

    def zero_body(j, _):
        sl = pl.ds(off + j * _L, _L)
        m_e[sl] = zeros_f
        m_n[sl] = zeros_f
        m_m[sl] = zeros_f
        return 0

    lax.fori_loop(0, _OUT_PER_SUB // _L, zero_body, 0, unroll=False)

    lane2 = lax.iota(jnp.int32, _L) * 2
    one_i = jnp.ones((_L,), jnp.int32)
    one_f = jnp.ones((_L,), jnp.float32)
    cap = jnp.full((_L,), _GRID - 1, jnp.int32)

    cp_lm.wait()

    def make_scatter_body(mask, base):
        def scatter_body(i, _):
            xi = base + i * (2 * _L) + lane2
            xs = plsc.load_gather(lm_v, [xi])
            ys = plsc.load_gather(lm_v, [xi + one_i])
            r = jnp.minimum(ys >> 4, cap)
            c = jnp.minimum(xs >> 4, cap)
            bins = (r << 5) | c
            plsc.store_scatter(mask, [bins], one_f)
            return 0
        return scatter_body

    base = 0
    for n_pts, mask in zip(_REGION_PTS, (m_e, m_n, m_m)):
        lax.fori_loop(0, n_pts // _L, make_scatter_body(mask, base), 0,
                      unroll=False)
        base += 2 * n_pts

    for cp in cp_w:
        cp.wait()

    def combine_body(j, _):
        sl = pl.ds(off + j * _L, _L)
        wl = pl.ds(j * _L, _L)
        acc = (ew_v[wl] - one_f) * m_e[sl] + one_f
        acc = acc + (nw_v[wl] - one_f) * m_n[sl]
        out_v[wl] = acc + (mw_v[wl] - one_f) * m_m[sl]
        return 0

    lax.fori_loop(0, _OUT_PER_SUB // _L, combine_body, 0, unroll=False)

    pltpu.sync_copy(out_v, out_hbm.at[pl.ds(off, _OUT_PER_SUB)])


def kernel(eye_landmarks, nose_landmarks, mouth_landmarks,
           eye_weight, nose_weight, mouth_weight):
    lm = jnp.concatenate([eye_landmarks.astype(jnp.int32).reshape(-1),
                          nose_landmarks.astype(jnp.int32).reshape(-1),
                          mouth_landmarks.astype(jnp.int32).reshape(-1)])

    mesh = plsc.VectorSubcoreMesh(core_axis_name="c", subcore_axis_name="s",
                                  num_cores=1)
    run = pl.kernel(
        _sc_body,
        out_type=jax.ShapeDtypeStruct((_NPATCH,), jnp.float32),
        mesh=mesh,
        compiler_params=pltpu.CompilerParams(needs_layout_passes=False),
        scratch_types=[
            pltpu.VMEM((2 * sum(_REGION_PTS),), jnp.int32),
            pltpu.VMEM((_OUT_PER_SUB,), jnp.float32),
            pltpu.VMEM((_OUT_PER_SUB,), jnp.float32),
            pltpu.VMEM((_OUT_PER_SUB,), jnp.float32),
            pltpu.VMEM((_NPATCH,), jnp.float32),
            pltpu.VMEM((_NPATCH,), jnp.float32),
            pltpu.VMEM((_NPATCH,), jnp.float32),
            pltpu.VMEM((_OUT_PER_SUB,), jnp.float32),
            pltpu.SemaphoreType.DMA,
            pltpu.SemaphoreType.DMA,
        ],
    )
    return run(lm, eye_weight, nose_weight, mouth_weight)
